# baseline (speedup 1.0000x reference)
import functools

import jax
import jax.numpy as jnp
from jax import lax
from jax.experimental import pallas as pl
from jax.experimental.pallas import tpu as pltpu
from jax.experimental.pallas import tpu_sc as plsc

_BATCH = 16384
_EMBED = 32
_BN_EPS = 1e-3

_NC = 2
_NS = 16
_NW = _NC * _NS
_BPW = _BATCH // _NW
_CHUNK = 128
_NCHUNK = _BPW // _CHUNK
_GROUPS = _BPW // 16


def _sc_kernel_body(uidx_hbm, cidx_hbm, ut_hbm, ct_hbm, sc_hbm, be_hbm,
                    out_hbm,
                    uidx_v, cidx_v, urows_v, crows_v, sc_v, be_v, out_v,
                    sem):
    wid = lax.axis_index("s") * _NC + lax.axis_index("c")

    pltpu.sync_copy(uidx_hbm.at[pl.ds(wid * _NCHUNK, _NCHUNK)], uidx_v)
    pltpu.sync_copy(cidx_hbm.at[pl.ds(wid * _NCHUNK, _NCHUNK)], cidx_v)
    pltpu.sync_copy(sc_hbm, sc_v)
    pltpu.sync_copy(be_hbm, be_v)

    copies = []
    for j in range(_NCHUNK):
        dst = urows_v.at[pl.ds(j * _CHUNK, _CHUNK)]
        copies.append(pltpu.async_copy(ut_hbm.at[uidx_v.at[j]], dst, sem))
    for j in range(_NCHUNK):
        dst = crows_v.at[pl.ds(j * _CHUNK, _CHUNK)]
        copies.append(pltpu.async_copy(ct_hbm.at[cidx_v.at[j]], dst, sem))
    for cp in copies:
        cp.wait()

    lane = lax.iota(jnp.int32, 16)
    s_half = [sc_v[pl.ds(0, 16)], sc_v[pl.ds(16, 16)]]
    b_half = [be_v[pl.ds(0, 16)], be_v[pl.ds(16, 16)]]

    def group_body(g, carry):
        rows = lane + g * 16
        acc = jnp.zeros((16,), jnp.float32)
        for d in range(_EMBED):
            col = jnp.full((16,), d, jnp.int32)
            u = plsc.load_gather(urows_v, [rows, col])
            c = plsc.load_gather(crows_v, [rows, col])
            s_d = s_half[d // 16][d % 16]
            b_d = b_half[d // 16][d % 16]
            acc = acc + (u * s_d + b_d) * (c * s_d + b_d)
        out_v[pl.ds(g * 16, 16)] = acc
        return carry

    lax.fori_loop(0, _GROUPS, group_body, 0, unroll=False)

    pltpu.sync_copy(out_v, out_hbm.at[pl.ds(wid * _BPW, _BPW)])


@jax.jit
def _run(uidx, cidx, user_table, content_table, scale, beta):
    mesh = plsc.VectorSubcoreMesh(core_axis_name="c", subcore_axis_name="s")
    kern = functools.partial(
        pl.kernel,
        mesh=mesh,
        out_type=jax.ShapeDtypeStruct((_BATCH,), jnp.float32),
        scratch_types=[
            pltpu.VMEM((_NCHUNK, _CHUNK), jnp.int32),
            pltpu.VMEM((_NCHUNK, _CHUNK), jnp.int32),
            pltpu.VMEM((_BPW, _EMBED), jnp.float32),
            pltpu.VMEM((_BPW, _EMBED), jnp.float32),
            pltpu.VMEM((_EMBED,), jnp.float32),
            pltpu.VMEM((_EMBED,), jnp.float32),
            pltpu.VMEM((_BPW,), jnp.float32),
            pltpu.SemaphoreType.DMA,
        ],
        compiler_params=pltpu.CompilerParams(
            needs_layout_passes=False, use_tc_tiling_on_sc=False),
    )(_sc_kernel_body)
    return kern(uidx, cidx, user_table, content_table, scale, beta)


def kernel(user, content, user_table, content_table, gamma, beta):
    scale = gamma / jnp.sqrt(1.0 + _BN_EPS)
    uidx = user.reshape(_NW * _NCHUNK, _CHUNK).astype(jnp.int32)
    cidx = content.reshape(_NW * _NCHUNK, _CHUNK).astype(jnp.int32)
    out = _run(uidx, cidx, user_table, content_table, scale, beta)
    return out.reshape(_BATCH, 1)


# speedup vs baseline: 1.4409x; 1.4409x over previous
import functools

import jax
import jax.numpy as jnp
from jax import lax
from jax.experimental import pallas as pl
from jax.experimental.pallas import tpu as pltpu
from jax.experimental.pallas import tpu_sc as plsc

_BATCH = 16384
_EMBED = 32
_BN_EPS = 1e-3

_NC = 2
_NS = 16
_NW = _NC * _NS
_BPW = _BATCH // _NW
_GROUPS = _BPW // 16


def _sc_kernel_body(uidx_hbm, cidx_hbm, ut_hbm, ct_hbm, sc_hbm, be_hbm,
                    out_hbm,
                    uidx_v, cidx_v, urows_v, crows_v, sc_v, be_v, out_v,
                    sem):
    wid = lax.axis_index("s") * _NC + lax.axis_index("c")

    pltpu.sync_copy(uidx_hbm.at[pl.ds(wid, 1)], uidx_v)
    pltpu.sync_copy(cidx_hbm.at[pl.ds(wid, 1)], cidx_v)
    pltpu.sync_copy(sc_hbm, sc_v)
    pltpu.sync_copy(be_hbm, be_v)

    lane = lax.iota(jnp.int32, 16)
    s_half = [sc_v[pl.ds(0, 16)], sc_v[pl.ds(16, 16)]]
    b_half = [be_v[pl.ds(0, 16)], be_v[pl.ds(16, 16)]]

    def group_body(g, carry):
        base = g * 16
        uvec = uidx_v[0, pl.ds(base, 16)]
        cvec = cidx_v[0, pl.ds(base, 16)]
        copies = []
        for r in range(16):
            copies.append(pltpu.async_copy(
                ut_hbm.at[pl.ds(uvec[r], 1)],
                urows_v.at[pl.ds(r, 1)], sem))
            copies.append(pltpu.async_copy(
                ct_hbm.at[pl.ds(cvec[r], 1)],
                crows_v.at[pl.ds(r, 1)], sem))
        for cp in copies:
            cp.wait()

        acc = jnp.zeros((16,), jnp.float32)
        for d in range(_EMBED):
            col = jnp.full((16,), d, jnp.int32)
            u = plsc.load_gather(urows_v, [lane, col])
            c = plsc.load_gather(crows_v, [lane, col])
            s_d = s_half[d // 16][d % 16]
            b_d = b_half[d // 16][d % 16]
            acc = acc + (u * s_d + b_d) * (c * s_d + b_d)
        out_v[pl.ds(base, 16)] = acc
        return carry

    lax.fori_loop(0, _GROUPS, group_body, 0, unroll=False)

    pltpu.sync_copy(out_v, out_hbm.at[pl.ds(wid * _BPW, _BPW)])


@jax.jit
def _run(uidx, cidx, user_table, content_table, scale, beta):
    mesh = plsc.VectorSubcoreMesh(core_axis_name="c", subcore_axis_name="s")
    kern = functools.partial(
        pl.kernel,
        mesh=mesh,
        out_type=jax.ShapeDtypeStruct((_BATCH,), jnp.float32),
        scratch_types=[
            pltpu.VMEM((1, _BPW), jnp.int32),
            pltpu.VMEM((1, _BPW), jnp.int32),
            pltpu.VMEM((16, _EMBED), jnp.float32),
            pltpu.VMEM((16, _EMBED), jnp.float32),
            pltpu.VMEM((_EMBED,), jnp.float32),
            pltpu.VMEM((_EMBED,), jnp.float32),
            pltpu.VMEM((_BPW,), jnp.float32),
            pltpu.SemaphoreType.DMA,
        ],
        compiler_params=pltpu.CompilerParams(needs_layout_passes=False),
    )(_sc_kernel_body)
    return kern(uidx, cidx, user_table, content_table, scale, beta)


def kernel(user, content, user_table, content_table, gamma, beta):
    scale = gamma / jnp.sqrt(1.0 + _BN_EPS)
    uidx = user.reshape(_NW, _BPW).astype(jnp.int32)
    cidx = content.reshape(_NW, _BPW).astype(jnp.int32)
    out = _run(uidx, cidx, user_table, content_table, scale, beta)
    return out.reshape(_BATCH, 1)


# speedup vs baseline: 1.4755x; 1.0240x over previous
import functools

import jax
import jax.numpy as jnp
from jax import lax
from jax.experimental import pallas as pl
from jax.experimental.pallas import tpu as pltpu
from jax.experimental.pallas import tpu_sc as plsc

_BATCH = 16384
_EMBED = 32
_BN_EPS = 1e-3

_NC = 2
_NS = 16
_NW = _NC * _NS
_BPW = _BATCH // _NW
_GROUPS = _BPW // 16


def _sc_kernel_body(uidx_hbm, cidx_hbm, ut_hbm, ct_hbm, sc_hbm, be_hbm,
                    out_hbm,
                    uidx_v, cidx_v, urows_v, crows_v, sc_v, be_v, out_v,
                    sem):
    wid = lax.axis_index("s") * _NC + lax.axis_index("c")

    pltpu.sync_copy(uidx_hbm.at[pl.ds(wid, 1)], uidx_v)
    pltpu.sync_copy(cidx_hbm.at[pl.ds(wid, 1)], cidx_v)
    pltpu.sync_copy(sc_hbm, sc_v)
    pltpu.sync_copy(be_hbm, be_v)

    lane = lax.iota(jnp.int32, 16)
    s0 = sc_v[pl.ds(0, 16)]
    s1 = sc_v[pl.ds(16, 16)]
    b0 = be_v[pl.ds(0, 16)]
    b1 = be_v[pl.ds(16, 16)]

    def group_body(g, carry):
        base = g * 16
        uvec = uidx_v[0, pl.ds(base, 16)]
        cvec = cidx_v[0, pl.ds(base, 16)]
        copies = []
        for r in range(16):
            copies.append(pltpu.async_copy(
                ut_hbm.at[pl.ds(uvec[r], 1)],
                urows_v.at[pl.ds(r, 1)], sem))
            copies.append(pltpu.async_copy(
                ct_hbm.at[pl.ds(cvec[r], 1)],
                crows_v.at[pl.ds(r, 1)], sem))
        for cp in copies:
            cp.wait()

        acc = jnp.zeros((16,), jnp.float32)
        for r in range(16):
            u0 = urows_v[r, pl.ds(0, 16)] * s0 + b0
            u1 = urows_v[r, pl.ds(16, 16)] * s1 + b1
            c0 = crows_v[r, pl.ds(0, 16)] * s0 + b0
            c1 = crows_v[r, pl.ds(16, 16)] * s1 + b1
            t = u0 * c0 + u1 * c1
            parts = [t[i] for i in range(16)]
            while len(parts) > 1:
                parts = [parts[i] + parts[i + 1]
                         for i in range(0, len(parts), 2)]
            acc = jnp.where(lane == r, parts[0], acc)
        out_v[pl.ds(base, 16)] = acc
        return carry

    lax.fori_loop(0, _GROUPS, group_body, 0, unroll=False)

    pltpu.sync_copy(out_v, out_hbm.at[pl.ds(wid * _BPW, _BPW)])


@jax.jit
def _run(uidx, cidx, user_table, content_table, scale, beta):
    mesh = plsc.VectorSubcoreMesh(core_axis_name="c", subcore_axis_name="s")
    kern = functools.partial(
        pl.kernel,
        mesh=mesh,
        out_type=jax.ShapeDtypeStruct((_BATCH,), jnp.float32),
        scratch_types=[
            pltpu.VMEM((1, _BPW), jnp.int32),
            pltpu.VMEM((1, _BPW), jnp.int32),
            pltpu.VMEM((16, _EMBED), jnp.float32),
            pltpu.VMEM((16, _EMBED), jnp.float32),
            pltpu.VMEM((_EMBED,), jnp.float32),
            pltpu.VMEM((_EMBED,), jnp.float32),
            pltpu.VMEM((_BPW,), jnp.float32),
            pltpu.SemaphoreType.DMA,
        ],
    )(_sc_kernel_body)
    return kern(uidx, cidx, user_table, content_table, scale, beta)


def kernel(user, content, user_table, content_table, gamma, beta):
    scale = gamma / jnp.sqrt(1.0 + _BN_EPS)
    uidx = user.reshape(_NW, _BPW).astype(jnp.int32)
    cidx = content.reshape(_NW, _BPW).astype(jnp.int32)
    out = _run(uidx, cidx, user_table, content_table, scale, beta)
    return out.reshape(_BATCH, 1)
